# baseline (speedup 1.0000x reference)
import jax
import jax.numpy as jnp
from jax.experimental import pallas as pl
from jax.experimental.pallas import tpu as pltpu


def _mm_kernel(a_ref, b_ref, o_ref):
    o_ref[...] = jnp.dot(a_ref[...], b_ref[...],
                         preferred_element_type=jnp.float32)


def _xw_pass(x, wcat, bm):
    m, k = x.shape
    kh = wcat.shape[1]
    return pl.pallas_call(
        _mm_kernel,
        grid=(m // bm,),
        in_specs=[
            pl.BlockSpec((bm, k), lambda i: (i, 0)),
            pl.BlockSpec((k, kh), lambda i: (0, 0)),
        ],
        out_specs=pl.BlockSpec((bm, kh), lambda i: (i, 0)),
        out_shape=jax.ShapeDtypeStruct((m, kh), jnp.float32),
    )(x, wcat)


def _uv_kernel(adj_ref, t_ref, u1_ref, v2_ref, v3_ref, u_scr):
    p = pl.program_id(0)
    i = pl.program_id(1)
    bm = adj_ref.shape[0]
    nh = u1_ref.shape[1]
    blk = pl.ds(i * bm, bm)

    @pl.when(p == 0)
    def _pass1():
        u = jnp.dot(adj_ref[...], t_ref[...],
                    preferred_element_type=jnp.float32)
        u_scr[blk, :] = u
        u1_ref[...] = u[:, :nh]
        v2_ref[...] = u[:, nh:2 * nh]
        v3_ref[...] = u[:, 2 * nh:]

    @pl.when(p == 1)
    def _pass2():
        v = jnp.dot(adj_ref[...], u_scr[:, nh:],
                    preferred_element_type=jnp.float32)
        v2_ref[...] = v[:, :nh]
        v3_ref[...] = v[:, nh:]
        u1_ref[...] = u_scr[blk, :nh]


def _uv_pass(adj, t, bm):
    m, n = adj.shape
    kh = t.shape[1]
    nh = kh // 3
    return pl.pallas_call(
        _uv_kernel,
        grid=(2, m // bm),
        in_specs=[
            pl.BlockSpec((bm, n), lambda p, i: (i, 0)),
            pl.BlockSpec((n, kh), lambda p, i: (0, 0)),
        ],
        out_specs=[
            pl.BlockSpec((bm, nh), lambda p, i: (i, 0)),
            pl.BlockSpec((bm, nh), lambda p, i: (i, 0)),
            pl.BlockSpec((bm, nh), lambda p, i: (i, 0)),
        ],
        out_shape=[
            jax.ShapeDtypeStruct((m, nh), jnp.float32),
            jax.ShapeDtypeStruct((m, nh), jnp.float32),
            jax.ShapeDtypeStruct((m, nh), jnp.float32),
        ],
        scratch_shapes=[pltpu.VMEM((m, kh), jnp.float32)],
    )(adj, t)


def _final_kernel(adj_ref, v3_ref, u1_ref, v2_ref, bcat_ref, wfc_ref,
                  bfc_ref, o_ref):
    w3 = jnp.dot(adj_ref[...], v3_ref[...],
                 preferred_element_type=jnp.float32)
    h = jnp.concatenate([u1_ref[...], v2_ref[...], w3], axis=1)
    h = jax.nn.relu(h + bcat_ref[...])
    logits = jnp.dot(h, wfc_ref[...], preferred_element_type=jnp.float32)
    o_ref[...] = jax.nn.sigmoid(logits + bfc_ref[...])


def _final_pass(adj, v3, u1, v2, bcat, wfc, bfc, bm):
    m, n = adj.shape
    k = v3.shape[1]
    kh = bcat.shape[1]
    nl = wfc.shape[1]
    return pl.pallas_call(
        _final_kernel,
        grid=(m // bm,),
        in_specs=[
            pl.BlockSpec((bm, n), lambda i: (i, 0)),
            pl.BlockSpec((n, k), lambda i: (0, 0)),
            pl.BlockSpec((bm, k), lambda i: (i, 0)),
            pl.BlockSpec((bm, k), lambda i: (i, 0)),
            pl.BlockSpec((1, kh), lambda i: (0, 0)),
            pl.BlockSpec((kh, nl), lambda i: (0, 0)),
            pl.BlockSpec((1, nl), lambda i: (0, 0)),
        ],
        out_specs=pl.BlockSpec((bm, nl), lambda i: (i, 0)),
        out_shape=jax.ShapeDtypeStruct((m, nl), jnp.float32),
    )(adj, v3, u1, v2, bcat, wfc, bfc)


def _pick_bm(m, cap):
    for bm in (400, 200, 80, 40, 16, 8):
        if bm <= cap and m % bm == 0:
            return bm
    return m


def kernel(x, adj, W1, b1, W2, b2, W3, b3, Wfc, bfc):
    m = adj.shape[0]

    wcat = jnp.concatenate([W1, W2, W3], axis=1)
    bcat = jnp.concatenate([b1, b2, b3])[None, :]

    t = _xw_pass(x, wcat, _pick_bm(m, 400))
    u1, v2, v3 = _uv_pass(adj, t, _pick_bm(m, 200))
    out = _final_pass(adj, v3, u1, v2, bcat, Wfc, bfc[None, :],
                      _pick_bm(m, 400))
    return out


# speedup vs baseline: 1.0339x; 1.0339x over previous
import jax
import jax.numpy as jnp
from jax.experimental import pallas as pl
from jax.experimental.pallas import tpu as pltpu


def _mm_kernel(a_ref, b_ref, o_ref):
    o_ref[...] = jnp.dot(a_ref[...], b_ref[...],
                         preferred_element_type=jnp.float32)


def _xw_pass(x, wcat, bm):
    m, k = x.shape
    kh = wcat.shape[1]
    return pl.pallas_call(
        _mm_kernel,
        grid=(m // bm,),
        in_specs=[
            pl.BlockSpec((bm, k), lambda i: (i, 0)),
            pl.BlockSpec((k, kh), lambda i: (0, 0)),
        ],
        out_specs=pl.BlockSpec((bm, kh), lambda i: (i, 0)),
        out_shape=jax.ShapeDtypeStruct((m, kh), jnp.float32),
    )(x, wcat)


def _mm_split_kernel(a_ref, b_ref, o1_ref, o2_ref):
    prod = jnp.dot(a_ref[...], b_ref[...], preferred_element_type=jnp.float32)
    k1 = o1_ref.shape[1]
    o1_ref[...] = prod[:, :k1]
    o2_ref[...] = prod[:, k1:]


def _pass1(adj, t, k1, bm):
    m, n = adj.shape
    kh = t.shape[1]
    return pl.pallas_call(
        _mm_split_kernel,
        grid=(m // bm,),
        in_specs=[
            pl.BlockSpec((bm, n), lambda i: (i, 0)),
            pl.BlockSpec((n, kh), lambda i: (0, 0)),
        ],
        out_specs=[
            pl.BlockSpec((bm, k1), lambda i: (i, 0)),
            pl.BlockSpec((bm, kh - k1), lambda i: (i, 0)),
        ],
        out_shape=[
            jax.ShapeDtypeStruct((m, k1), jnp.float32),
            jax.ShapeDtypeStruct((m, kh - k1), jnp.float32),
        ],
    )(adj, t)


def _vw_kernel(adj_ref, u23_ref, u1_ref, bcat_ref, wfc_ref, bfc_ref,
               o_ref, v_scr):
    p = pl.program_id(0)
    i = pl.program_id(1)
    bm = adj_ref.shape[0]
    nh = u1_ref.shape[1]
    blk = pl.ds(i * bm, bm)

    @pl.when(p == 0)
    def _pass2():
        v_scr[blk, :] = jnp.dot(adj_ref[...], u23_ref[...],
                                preferred_element_type=jnp.float32)
        o_ref[...] = jnp.zeros_like(o_ref)

    @pl.when(p == 1)
    def _pass3_epilogue():
        w3 = jnp.dot(adj_ref[...], v_scr[:, nh:],
                     preferred_element_type=jnp.float32)
        h = jnp.concatenate([u1_ref[...], v_scr[blk, :nh], w3], axis=1)
        h = jax.nn.relu(h + bcat_ref[...])
        logits = jnp.dot(h, wfc_ref[...], preferred_element_type=jnp.float32)
        o_ref[...] = jax.nn.sigmoid(logits + bfc_ref[...])


def _vw_pass(adj, u23, u1, bcat, wfc, bfc, bm):
    m, n = adj.shape
    k2 = u23.shape[1]
    nh = u1.shape[1]
    kh = bcat.shape[1]
    nl = wfc.shape[1]
    return pl.pallas_call(
        _vw_kernel,
        grid=(2, m // bm),
        in_specs=[
            pl.BlockSpec((bm, n), lambda p, i: (i, 0)),
            pl.BlockSpec((n, k2), lambda p, i: (0, 0)),
            pl.BlockSpec((bm, nh), lambda p, i: (i, 0)),
            pl.BlockSpec((1, kh), lambda p, i: (0, 0)),
            pl.BlockSpec((kh, nl), lambda p, i: (0, 0)),
            pl.BlockSpec((1, nl), lambda p, i: (0, 0)),
        ],
        out_specs=pl.BlockSpec((bm, nl), lambda p, i: (i, 0)),
        out_shape=jax.ShapeDtypeStruct((m, nl), jnp.float32),
        scratch_shapes=[pltpu.VMEM((m, k2), jnp.float32)],
    )(adj, u23, u1, bcat, wfc, bfc)


def _pick_bm(m, cap):
    for bm in (400, 200, 80, 40, 16, 8):
        if bm <= cap and m % bm == 0:
            return bm
    return m


def kernel(x, adj, W1, b1, W2, b2, W3, b3, Wfc, bfc):
    m = adj.shape[0]
    nh = W1.shape[1]
    bm = _pick_bm(m, 400)

    wcat = jnp.concatenate([W1, W2, W3], axis=1)
    bcat = jnp.concatenate([b1, b2, b3])[None, :]

    t = _xw_pass(x, wcat, bm)
    u1, u23 = _pass1(adj, t, nh, bm)
    out = _vw_pass(adj, u23, u1, bcat, Wfc, bfc[None, :], bm)
    return out


# speedup vs baseline: 1.0831x; 1.0476x over previous
import jax
import jax.numpy as jnp
from jax.experimental import pallas as pl
from jax.experimental.pallas import tpu as pltpu


def _tu_kernel(adj_ref, x_ref, wcat_ref, o1_ref, o2_ref, t_scr):
    @pl.when(pl.program_id(0) == 0)
    def _stage_t():
        t_scr[...] = jnp.dot(x_ref[...], wcat_ref[...],
                             preferred_element_type=jnp.float32)

    prod = jnp.dot(adj_ref[...], t_scr[...],
                   preferred_element_type=jnp.float32)
    k1 = o1_ref.shape[1]
    o1_ref[...] = prod[:, :k1]
    o2_ref[...] = prod[:, k1:]


def _pass1(adj, x, wcat, k1, bm):
    m, n = adj.shape
    k = x.shape[1]
    kh = wcat.shape[1]
    return pl.pallas_call(
        _tu_kernel,
        grid=(m // bm,),
        in_specs=[
            pl.BlockSpec((bm, n), lambda i: (i, 0)),
            pl.BlockSpec((n, k), lambda i: (0, 0)),
            pl.BlockSpec((k, kh), lambda i: (0, 0)),
        ],
        out_specs=[
            pl.BlockSpec((bm, k1), lambda i: (i, 0)),
            pl.BlockSpec((bm, kh - k1), lambda i: (i, 0)),
        ],
        out_shape=[
            jax.ShapeDtypeStruct((m, k1), jnp.float32),
            jax.ShapeDtypeStruct((m, kh - k1), jnp.float32),
        ],
        scratch_shapes=[pltpu.VMEM((n, kh), jnp.float32)],
    )(adj, x, wcat)


def _vw_kernel(adj_ref, u23_ref, u1_ref, bcat_ref, wfc_ref, bfc_ref,
               o_ref, v_scr):
    p = pl.program_id(0)
    i = pl.program_id(1)
    bm = adj_ref.shape[0]
    nh = u1_ref.shape[1]
    blk = pl.ds(i * bm, bm)

    @pl.when(p == 0)
    def _pass2():
        v_scr[blk, :] = jnp.dot(adj_ref[...], u23_ref[...],
                                preferred_element_type=jnp.float32)
        o_ref[...] = jnp.zeros_like(o_ref)

    @pl.when(p == 1)
    def _pass3_epilogue():
        w3 = jnp.dot(adj_ref[...], v_scr[:, nh:],
                     preferred_element_type=jnp.float32)
        h = jnp.concatenate([u1_ref[...], v_scr[blk, :nh], w3], axis=1)
        h = jax.nn.relu(h + bcat_ref[...])
        logits = jnp.dot(h, wfc_ref[...], preferred_element_type=jnp.float32)
        o_ref[...] = jax.nn.sigmoid(logits + bfc_ref[...])


def _vw_pass(adj, u23, u1, bcat, wfc, bfc, bm):
    m, n = adj.shape
    k2 = u23.shape[1]
    nh = u1.shape[1]
    kh = bcat.shape[1]
    nl = wfc.shape[1]
    return pl.pallas_call(
        _vw_kernel,
        grid=(2, m // bm),
        in_specs=[
            pl.BlockSpec((bm, n), lambda p, i: (i, 0)),
            pl.BlockSpec((n, k2), lambda p, i: (0, 0)),
            pl.BlockSpec((bm, nh), lambda p, i: (i, 0)),
            pl.BlockSpec((1, kh), lambda p, i: (0, 0)),
            pl.BlockSpec((kh, nl), lambda p, i: (0, 0)),
            pl.BlockSpec((1, nl), lambda p, i: (0, 0)),
        ],
        out_specs=pl.BlockSpec((bm, nl), lambda p, i: (i, 0)),
        out_shape=jax.ShapeDtypeStruct((m, nl), jnp.float32),
        scratch_shapes=[pltpu.VMEM((m, k2), jnp.float32)],
    )(adj, u23, u1, bcat, wfc, bfc)


def _pick_bm(m, cap):
    for bm in (400, 200, 80, 40, 16, 8):
        if bm <= cap and m % bm == 0:
            return bm
    return m


def kernel(x, adj, W1, b1, W2, b2, W3, b3, Wfc, bfc):
    m = adj.shape[0]
    nh = W1.shape[1]
    bm = _pick_bm(m, 400)

    wcat = jnp.concatenate([W1, W2, W3], axis=1)
    bcat = jnp.concatenate([b1, b2, b3])[None, :]

    u1, u23 = _pass1(adj, x, wcat, nh, bm)
    out = _vw_pass(adj, u23, u1, bcat, Wfc, bfc[None, :], bm)
    return out
